# Initial kernel scaffold; baseline (speedup 1.0000x reference)
import jax
import jax.numpy as jnp
from jax.experimental import pallas as pl


def kernel(xyz, new_xyz):
    raise NotImplementedError("write your pallas kernel here")


# speedup vs baseline: 17.1986x; 17.1986x over previous
import functools

import jax
import jax.numpy as jnp
from jax import lax
from jax.experimental import pallas as pl
from jax.experimental.pallas import tpu as pltpu
from jax.experimental.pallas import tpu_sc as plsc

_RADIUS2 = 0.2 * 0.2
_NSAMPLE = 64
_L = 16
_UNROLL = 8


def _ball_query_body(nbatch, xyz_hbm, q_hbm, out_hbm,
                     xk_v, yk_v, zk_v, qx_v, qy_v, qz_v, buf_v, out_v):
    n = xyz_hbm.shape[0] // (3 * nbatch)
    m = q_hbm.shape[0] // (3 * nbatch)
    sentinel = n + 1
    w = lax.axis_index("s") * 2 + lax.axis_index("c")
    workers_per_batch = 32 // nbatch
    qs_per_worker = m // workers_per_batch
    b = w // workers_per_batch
    q0 = (w % workers_per_batch) * qs_per_worker

    pltpu.sync_copy(xyz_hbm.at[pl.ds((b * 3 + 0) * n, n)], xk_v)
    pltpu.sync_copy(xyz_hbm.at[pl.ds((b * 3 + 1) * n, n)], yk_v)
    pltpu.sync_copy(xyz_hbm.at[pl.ds((b * 3 + 2) * n, n)], zk_v)
    pltpu.sync_copy(q_hbm.at[pl.ds((b * 3 + 0) * m, m)], qx_v)
    pltpu.sync_copy(q_hbm.at[pl.ds((b * 3 + 1) * m, m)], qy_v)
    pltpu.sync_copy(q_hbm.at[pl.ds((b * 3 + 2) * m, m)], qz_v)

    lanes = lax.broadcasted_iota(jnp.int32, (_L,), 0)
    zeros = jnp.zeros((_L,), jnp.int32)
    num_chunks = n // _L

    def per_query(i, carry):
        qi = jnp.full((_L,), q0 + i, jnp.int32)
        qx = plsc.load_gather(qx_v, [qi])
        qy = plsc.load_gather(qy_v, [qi])
        qz = plsc.load_gather(qz_v, [qi])
        buf_v[pl.ds(0, _L)] = jnp.full((_L,), sentinel, jnp.int32)

        def cond(c):
            chunk, cnt = c
            return (cnt < _NSAMPLE) & (chunk < num_chunks)

        def scan_step(c):
            chunk, cnt = c
            cnt_v = jnp.full((_L,), cnt, jnp.int32)
            for u in range(_UNROLL):
                base = pl.multiple_of((chunk + u) * _L, _L)
                xk = xk_v[pl.ds(base, _L)]
                yk = yk_v[pl.ds(base, _L)]
                zk = zk_v[pl.ds(base, _L)]
                dx = xk - qx
                dy = yk - qy
                dz = zk - qz
                d2 = dx * dx + dy * dy + dz * dz
                msk = d2 < jnp.float32(_RADIUS2)
                pos = cnt_v + plsc.cumsum(msk.astype(jnp.int32)) - 1
                plsc.store_scatter(buf_v, [pos], base + lanes, mask=msk)
                cnt_v = cnt_v + plsc.all_reduce_population_count(msk)
            return (chunk + _UNROLL, jnp.max(cnt_v))

        _, cnt = lax.while_loop(cond, scan_step, (jnp.int32(0), jnp.int32(0)))

        cnt_v = jnp.full((_L,), cnt, jnp.int32)
        vals0 = buf_v[pl.ds(0, _L)]
        first = jnp.take_along_axis(vals0, zeros, axis=0)
        for j in range(_NSAMPLE // _L):
            vals = vals0 if j == 0 else buf_v[pl.ds(j * _L, _L)]
            valid = (lanes + j * _L) < cnt_v
            off = pl.multiple_of(i * _NSAMPLE + j * _L, _L)
            out_v[pl.ds(off, _L)] = jnp.where(valid, vals, first)
        return carry

    lax.fori_loop(0, qs_per_worker, per_query, 0)
    pltpu.sync_copy(out_v, out_hbm.at[pl.ds(w * qs_per_worker * _NSAMPLE,
                                            qs_per_worker * _NSAMPLE)])


def kernel(xyz, new_xyz):
    b, m, _ = new_xyz.shape
    n = xyz.shape[1]
    qs_per_worker = (b * m) // 32
    xyz_t = jnp.transpose(xyz, (0, 2, 1)).reshape(-1)
    q_t = jnp.transpose(new_xyz, (0, 2, 1)).reshape(-1)

    run = pl.kernel(
        functools.partial(_ball_query_body, b),
        out_type=jax.ShapeDtypeStruct((b * m * _NSAMPLE,), jnp.int32),
        mesh=plsc.VectorSubcoreMesh(core_axis_name="c", subcore_axis_name="s"),
        compiler_params=pltpu.CompilerParams(needs_layout_passes=False),
        scratch_types=[
            pltpu.VMEM((n,), jnp.float32),
            pltpu.VMEM((n,), jnp.float32),
            pltpu.VMEM((n,), jnp.float32),
            pltpu.VMEM((m,), jnp.float32),
            pltpu.VMEM((m,), jnp.float32),
            pltpu.VMEM((m,), jnp.float32),
            pltpu.VMEM((224,), jnp.int32),
            pltpu.VMEM((qs_per_worker * _NSAMPLE,), jnp.int32),
        ],
    )
    out = run(xyz_t, q_t)
    return out.reshape(b, m, _NSAMPLE)


# speedup vs baseline: 33.9168x; 1.9721x over previous
import functools

import jax
import jax.numpy as jnp
from jax import lax
from jax.experimental import pallas as pl
from jax.experimental.pallas import tpu as pltpu
from jax.experimental.pallas import tpu_sc as plsc

_RADIUS2 = 0.2 * 0.2
_NSAMPLE = 64
_L = 16
_UNROLL = 8
_QGROUP = 4
_BUFSZ = 80


def _ball_query_body(nbatch, xyz_hbm, q_hbm, out_hbm,
                     xk_v, yk_v, zk_v, qx_v, qy_v, qz_v, buf_v, out_v):
    n = xyz_hbm.shape[0] // (3 * nbatch)
    m = q_hbm.shape[0] // (3 * nbatch)
    sentinel = n + 1
    w = lax.axis_index("s") * 2 + lax.axis_index("c")
    workers_per_batch = 32 // nbatch
    qs_per_worker = m // workers_per_batch
    b = w // workers_per_batch
    q0 = (w % workers_per_batch) * qs_per_worker

    pltpu.sync_copy(xyz_hbm.at[pl.ds((b * 3 + 0) * n, n)], xk_v)
    pltpu.sync_copy(xyz_hbm.at[pl.ds((b * 3 + 1) * n, n)], yk_v)
    pltpu.sync_copy(xyz_hbm.at[pl.ds((b * 3 + 2) * n, n)], zk_v)
    pltpu.sync_copy(q_hbm.at[pl.ds((b * 3 + 0) * m, m)], qx_v)
    pltpu.sync_copy(q_hbm.at[pl.ds((b * 3 + 1) * m, m)], qy_v)
    pltpu.sync_copy(q_hbm.at[pl.ds((b * 3 + 2) * m, m)], qz_v)

    lanes = lax.broadcasted_iota(jnp.int32, (_L,), 0)
    zeros = jnp.zeros((_L,), jnp.int32)
    num_chunks = n // _L

    limit = jnp.full((_L,), _NSAMPLE, jnp.int32)

    def per_group(g, carry):
        qxs, qys, qzs = [], [], []
        for q in range(_QGROUP):
            qi = jnp.full((_L,), q0 + g * _QGROUP + q, jnp.int32)
            qxs.append(plsc.load_gather(qx_v, [qi]))
            qys.append(plsc.load_gather(qy_v, [qi]))
            qzs.append(plsc.load_gather(qz_v, [qi]))
            buf_v[pl.ds(q * _BUFSZ, _L)] = jnp.full((_L,), sentinel, jnp.int32)

        def cond(c):
            chunk, cnts = c
            mn = jnp.minimum(jnp.minimum(cnts[0], cnts[1]),
                             jnp.minimum(cnts[2], cnts[3]))
            return (mn[0] < _NSAMPLE) & (chunk < num_chunks)

        def scan_step(c):
            chunk, cnts = c
            cnts = list(cnts)
            for u in range(_UNROLL):
                base = pl.multiple_of((chunk + u) * _L, _L)
                xk = xk_v[pl.ds(base, _L)]
                yk = yk_v[pl.ds(base, _L)]
                zk = zk_v[pl.ds(base, _L)]
                idx = base + lanes
                for q in range(_QGROUP):
                    dx = xk - qxs[q]
                    dy = yk - qys[q]
                    dz = zk - qzs[q]
                    d2 = dx * dx + dy * dy + dz * dz
                    msk = d2 < jnp.float32(_RADIUS2)
                    pos = cnts[q] + plsc.cumsum(msk.astype(jnp.int32)) - 1
                    pos = jnp.minimum(pos, limit) + (q * _BUFSZ)
                    plsc.store_scatter(buf_v, [pos], idx, mask=msk)
                    cnts[q] = cnts[q] + plsc.all_reduce_population_count(msk)
            return (chunk + _UNROLL, tuple(cnts))

        _, cnts = lax.while_loop(
            cond, scan_step, (jnp.int32(0), (zeros, zeros, zeros, zeros)))

        for q in range(_QGROUP):
            cnt_v = cnts[q]
            vals0 = buf_v[pl.ds(q * _BUFSZ, _L)]
            first = jnp.take_along_axis(vals0, zeros, axis=0)
            for j in range(_NSAMPLE // _L):
                vals = vals0 if j == 0 else buf_v[pl.ds(q * _BUFSZ + j * _L, _L)]
                valid = (lanes + j * _L) < cnt_v
                off = pl.multiple_of((g * _QGROUP + q) * _NSAMPLE + j * _L, _L)
                out_v[pl.ds(off, _L)] = jnp.where(valid, vals, first)
        return carry

    lax.fori_loop(0, qs_per_worker // _QGROUP, per_group, 0)
    pltpu.sync_copy(out_v, out_hbm.at[pl.ds(w * qs_per_worker * _NSAMPLE,
                                            qs_per_worker * _NSAMPLE)])


def kernel(xyz, new_xyz):
    b, m, _ = new_xyz.shape
    n = xyz.shape[1]
    qs_per_worker = (b * m) // 32
    xyz_t = jnp.transpose(xyz, (0, 2, 1)).reshape(-1)
    q_t = jnp.transpose(new_xyz, (0, 2, 1)).reshape(-1)

    run = pl.kernel(
        functools.partial(_ball_query_body, b),
        out_type=jax.ShapeDtypeStruct((b * m * _NSAMPLE,), jnp.int32),
        mesh=plsc.VectorSubcoreMesh(core_axis_name="c", subcore_axis_name="s"),
        compiler_params=pltpu.CompilerParams(needs_layout_passes=False),
        scratch_types=[
            pltpu.VMEM((n,), jnp.float32),
            pltpu.VMEM((n,), jnp.float32),
            pltpu.VMEM((n,), jnp.float32),
            pltpu.VMEM((m,), jnp.float32),
            pltpu.VMEM((m,), jnp.float32),
            pltpu.VMEM((m,), jnp.float32),
            pltpu.VMEM((_QGROUP * _BUFSZ,), jnp.int32),
            pltpu.VMEM((qs_per_worker * _NSAMPLE,), jnp.int32),
        ],
    )
    out = run(xyz_t, q_t)
    return out.reshape(b, m, _NSAMPLE)
